# baseline (speedup 1.0000x reference)
import functools

import jax
import jax.numpy as jnp
from jax import lax
from jax.experimental import pallas as pl
from jax.experimental.pallas import tpu as pltpu
from jax.experimental.pallas import tpu_sc as plsc

BATCH = 16384
DIM = 64
NC = 2
NS = 16
NW = NC * NS
ROWS_PER_W = BATCH // NW
CHUNK = 128
NCHUNK = ROWS_PER_W // CHUNK
GPC = CHUNK // 16

ENT = 1000000
EB = 256
NBLK = 999936 // EB
TAIL0 = 999936
SLOTS = 63

_MESH = plsc.VectorSubcoreMesh(core_axis_name="c", subcore_axis_name="s")
_PARAMS = pltpu.CompilerParams(needs_layout_passes=False,
                               disable_bounds_checks=True)


def _transpose_body(ent_t, ent_tail, out_hbm, abuf0, abuf1, tbuf0, tbuf1,
                    tailbuf, sem_in, sem_out):
    c = lax.axis_index("c")
    s = lax.axis_index("s")
    wid = s * NC + c
    nblk = jnp.where(wid < NBLK - 32 * (NBLK // NW), NBLK // NW + 1,
                     NBLK // NW)
    abufs = (abuf0, abuf1)
    tbufs = (tbuf0, tbuf1)
    iota16 = lax.iota(jnp.int32, 16)

    def e0_of(gi):
        return pl.multiple_of((wid + NW * gi) * EB, EB)

    def start_in(gi, buf):
        pltpu.async_copy(ent_t.at[:, pl.ds(e0_of(gi), EB)],
                         buf.at[:, pl.ds(0, EB)], sem_in)

    start_in(0, abuf0)
    start_in(1, abuf1)

    def slot_pair(ii, carry):
        for b in range(2):
            gi = 2 * ii + b
            ab, tb = abufs[b], tbufs[b]

            @pl.when(gi < nblk)
            def _work():
                pltpu.make_async_copy(ent_t.at[:, pl.ds(0, EB)],
                                      ab.at[:, pl.ds(0, EB)], sem_in).wait()

            @pl.when(jnp.logical_and(gi >= 2, gi - 2 < nblk))
            def _drain():
                pltpu.make_async_copy(tb, out_hbm.at[pl.ds(0, EB // 2)],
                                      sem_out).wait()

            @pl.when(gi < nblk)
            def _transpose():
                @plsc.parallel_loop(0, EB // 2, unroll=8)
                def _ploop(p):
                    ev0 = jnp.full((16,), 2 * p, jnp.int32)
                    ev1 = ev0 + 1
                    for k in range(8):
                        dvec = 16 * (k % 4) + iota16
                        ev = ev1 if k >= 4 else ev0
                        tb[p, pl.ds(16 * k, 16)] = (
                            plsc.load_gather(ab, [dvec, ev]))
                pltpu.async_copy(
                    tb, out_hbm.at[pl.ds(
                        pl.multiple_of((wid + NW * gi) * (EB // 2), EB // 2),
                        EB // 2)],
                    sem_out)

            @pl.when(gi + 2 < nblk)
            def _next_in():
                start_in(gi + 2, ab)

        return carry

    lax.fori_loop(0, SLOTS, slot_pair, 0)

    @pl.when(wid == 2)
    def _tail():
        pltpu.sync_copy(ent_tail, tailbuf)
        for p in range(32):
            ev0 = jnp.full((16,), 2 * p, jnp.int32)
            ev1 = jnp.full((16,), 2 * p + 1, jnp.int32)
            for k in range(8):
                dvec = 16 * (k % 4) + iota16
                ev = ev1 if k >= 4 else ev0
                tbuf0[p, pl.ds(16 * k, 16)] = plsc.load_gather(tailbuf,
                                                               [dvec, ev])
        pltpu.sync_copy(tbuf0.at[pl.ds(0, 32)],
                        out_hbm.at[pl.ds(TAIL0 // 2, 32)])


@functools.partial(
    pl.kernel,
    mesh=_MESH,
    out_type=jax.ShapeDtypeStruct((ENT // 2, 2 * DIM), jnp.float32),
    compiler_params=_PARAMS,
    scratch_types=[
        pltpu.VMEM((DIM, EB + 1), jnp.float32),
        pltpu.VMEM((DIM, EB + 1), jnp.float32),
        pltpu.VMEM((EB // 2, 2 * DIM), jnp.float32),
        pltpu.VMEM((EB // 2, 2 * DIM), jnp.float32),
        pltpu.VMEM((DIM, 64), jnp.float32),
        pltpu.SemaphoreType.DMA,
        pltpu.SemaphoreType.DMA,
    ],
)
def _transpose_sc(*args):
    _transpose_body(*args)


def _gather_body(head_hbm, tail_hbm, rel_hbm, ent_hbm, relemb_hbm, out_hbm,
                 hidx, tidx, ridx, gidx, hbuf, tbuf, rbuf, oscr, sem):
    c = lax.axis_index("c")
    s = lax.axis_index("s")
    wid = s * NC + c
    base = wid * ROWS_PER_W

    pltpu.sync_copy(head_hbm.at[pl.ds(base, ROWS_PER_W)], hidx)
    pltpu.sync_copy(tail_hbm.at[pl.ds(base, ROWS_PER_W)], tidx)
    pltpu.sync_copy(rel_hbm.at[pl.ds(base, ROWS_PER_W)], ridx)

    iota16 = lax.iota(jnp.int32, 16)

    def chunk_body(ck, carry):
        off = pl.multiple_of(ck * CHUNK, CHUNK)
        for v in range(CHUNK // 16):
            gidx[pl.ds(16 * v, 16)] = (
                lax.shift_right_logical(hidx[pl.ds(off + 16 * v, 16)], 1))
        cph = pltpu.async_copy(ent_hbm.at[gidx], hbuf, sem)
        cph.wait()
        for v in range(CHUNK // 16):
            gidx[pl.ds(16 * v, 16)] = (
                lax.shift_right_logical(tidx[pl.ds(off + 16 * v, 16)], 1))
        cpt = pltpu.async_copy(ent_hbm.at[gidx], tbuf, sem)
        cpt.wait()
        for v in range(CHUNK // 16):
            gidx[pl.ds(16 * v, 16)] = (
                lax.shift_right_logical(ridx[pl.ds(off + 16 * v, 16)], 1))
        cpr = pltpu.async_copy(relemb_hbm.at[gidx], rbuf, sem)
        cpr.wait()

        for g in range(GPC):
            goff = off + g * 16
            slot = g * 16 + iota16
            hsel = (hidx[pl.ds(goff, 16)] & 1) * 64
            tsel = (tidx[pl.ds(goff, 16)] & 1) * 64
            rsel = (ridx[pl.ds(goff, 16)] & 1) * 64
            acc = jnp.zeros((16,), jnp.float32)
            for d in range(DIM):
                rot = (jnp.full((16,), d, jnp.int32) + iota16) & 63
                h = plsc.load_gather(hbuf, [slot, hsel + rot])
                t = plsc.load_gather(tbuf, [slot, tsel + rot])
                r = plsc.load_gather(rbuf, [slot, rsel + rot])
                acc = acc + h * t * r
            score = 1.0 / (1.0 + jnp.exp(-acc))
            oscr[pl.ds(goff, 16)] = score
        return carry

    lax.fori_loop(0, NCHUNK, chunk_body, 0)

    pltpu.sync_copy(oscr, out_hbm.at[pl.ds(base, ROWS_PER_W)])


@functools.partial(
    pl.kernel,
    mesh=_MESH,
    out_type=jax.ShapeDtypeStruct((BATCH,), jnp.float32),
    compiler_params=_PARAMS,
    scratch_types=[
        pltpu.VMEM((ROWS_PER_W,), jnp.int32),
        pltpu.VMEM((ROWS_PER_W,), jnp.int32),
        pltpu.VMEM((ROWS_PER_W,), jnp.int32),
        pltpu.VMEM((CHUNK,), jnp.int32),
        pltpu.VMEM((CHUNK, 2 * DIM), jnp.float32),
        pltpu.VMEM((CHUNK, 2 * DIM), jnp.float32),
        pltpu.VMEM((CHUNK, 2 * DIM), jnp.float32),
        pltpu.VMEM((ROWS_PER_W,), jnp.float32),
        pltpu.SemaphoreType.DMA,
    ],
)
def _gather_sc(*args):
    _gather_body(*args)


def kernel(head, tail, relation, entity_embed, relation_embed):
    ent2 = _transpose_sc(entity_embed.T, entity_embed[TAIL0:].T)
    rel2 = relation_embed.reshape(relation_embed.shape[0] // 2, 2 * DIM)
    return _gather_sc(head.astype(jnp.int32), tail.astype(jnp.int32),
                      relation.astype(jnp.int32), ent2, rel2)


# speedup vs baseline: 1.7898x; 1.7898x over previous
import functools

import jax
import jax.numpy as jnp
from jax import lax
from jax.experimental import pallas as pl
from jax.experimental.pallas import tpu as pltpu
from jax.experimental.pallas import tpu_sc as plsc

BATCH = 16384
DIM = 64
NC = 2
NS = 16
NW = NC * NS
ROWS_PER_W = BATCH // NW
CHUNK = 128
NCHUNK = ROWS_PER_W // CHUNK
GPC = CHUNK // 16

ENT = 1000000
EB = 256
NBLK = 999936 // EB
TAIL0 = 999936
SLOTS = 63

_MESH = plsc.VectorSubcoreMesh(core_axis_name="c", subcore_axis_name="s")
_PARAMS = pltpu.CompilerParams(needs_layout_passes=False,
                               disable_bounds_checks=True)


def _transpose_body(ent_t, ent_tail, out_hbm, abuf0, abuf1, tbuf0, tbuf1,
                    tailbuf, sem_in, sem_out):
    c = lax.axis_index("c")
    s = lax.axis_index("s")
    wid = s * NC + c
    nblk = jnp.where(wid < NBLK - 32 * (NBLK // NW), NBLK // NW + 1,
                     NBLK // NW)
    abufs = (abuf0, abuf1)
    tbufs = (tbuf0, tbuf1)
    iota16 = lax.iota(jnp.int32, 16)

    def e0_of(gi):
        return pl.multiple_of((wid + NW * gi) * EB, EB)

    def start_in(gi, buf):
        pltpu.async_copy(ent_t.at[:, pl.ds(e0_of(gi), EB)],
                         buf.at[:, pl.ds(0, EB)], sem_in)

    start_in(0, abuf0)
    start_in(1, abuf1)

    def slot_pair(ii, carry):
        for b in range(2):
            gi = 2 * ii + b
            ab, tb = abufs[b], tbufs[b]

            @pl.when(gi < nblk)
            def _work():
                pltpu.make_async_copy(ent_t.at[:, pl.ds(0, EB)],
                                      ab.at[:, pl.ds(0, EB)], sem_in).wait()

            @pl.when(jnp.logical_and(gi >= 2, gi - 2 < nblk))
            def _drain():
                pltpu.make_async_copy(tb, out_hbm.at[pl.ds(0, EB // 2)],
                                      sem_out).wait()

            @pl.when(gi < nblk)
            def _transpose():
                @plsc.parallel_loop(0, EB, step=16, unroll=2)
                def _ploop(x0):
                    xv = x0 + iota16
                    pv = lax.shift_right_logical(xv, 1)
                    sel = (xv & 1) * 64
                    for r in range(16):
                        rot = (iota16 + r) & 15
                        for d0 in range(0, DIM, 16):
                            dv = d0 + rot
                            v = plsc.load_gather(ab, [dv, xv])
                            plsc.store_scatter(tb, [pv, sel + dv], v)
                pltpu.async_copy(
                    tb, out_hbm.at[pl.ds(
                        pl.multiple_of((wid + NW * gi) * (EB // 2), EB // 2),
                        EB // 2)],
                    sem_out)

            @pl.when(gi + 2 < nblk)
            def _next_in():
                start_in(gi + 2, ab)

        return carry

    lax.fori_loop(0, SLOTS, slot_pair, 0)

    @pl.when(wid == 2)
    def _tail():
        pltpu.sync_copy(ent_tail, tailbuf)
        for p in range(32):
            ev0 = jnp.full((16,), 2 * p, jnp.int32)
            ev1 = jnp.full((16,), 2 * p + 1, jnp.int32)
            for k in range(8):
                dvec = 16 * (k % 4) + iota16
                ev = ev1 if k >= 4 else ev0
                tbuf0[p, pl.ds(16 * k, 16)] = plsc.load_gather(tailbuf,
                                                               [dvec, ev])
        pltpu.sync_copy(tbuf0.at[pl.ds(0, 32)],
                        out_hbm.at[pl.ds(TAIL0 // 2, 32)])


@functools.partial(
    pl.kernel,
    mesh=_MESH,
    out_type=jax.ShapeDtypeStruct((ENT // 2, 2 * DIM), jnp.float32),
    compiler_params=_PARAMS,
    scratch_types=[
        pltpu.VMEM((DIM, EB + 1), jnp.float32),
        pltpu.VMEM((DIM, EB + 1), jnp.float32),
        pltpu.VMEM((EB // 2, 2 * DIM), jnp.float32),
        pltpu.VMEM((EB // 2, 2 * DIM), jnp.float32),
        pltpu.VMEM((DIM, 64), jnp.float32),
        pltpu.SemaphoreType.DMA,
        pltpu.SemaphoreType.DMA,
    ],
)
def _transpose_sc(*args):
    _transpose_body(*args)


def _gather_body(head_hbm, tail_hbm, rel_hbm, ent_hbm, relemb_hbm, out_hbm,
                 hidx, tidx, ridx, gidx, hbuf, tbuf, rbuf, oscr, sem):
    c = lax.axis_index("c")
    s = lax.axis_index("s")
    wid = s * NC + c
    base = wid * ROWS_PER_W

    pltpu.sync_copy(head_hbm.at[pl.ds(base, ROWS_PER_W)], hidx)
    pltpu.sync_copy(tail_hbm.at[pl.ds(base, ROWS_PER_W)], tidx)
    pltpu.sync_copy(rel_hbm.at[pl.ds(base, ROWS_PER_W)], ridx)

    iota16 = lax.iota(jnp.int32, 16)

    def chunk_body(ck, carry):
        off = pl.multiple_of(ck * CHUNK, CHUNK)
        for v in range(CHUNK // 16):
            gidx[pl.ds(16 * v, 16)] = (
                lax.shift_right_logical(hidx[pl.ds(off + 16 * v, 16)], 1))
        cph = pltpu.async_copy(ent_hbm.at[gidx], hbuf, sem)
        cph.wait()
        for v in range(CHUNK // 16):
            gidx[pl.ds(16 * v, 16)] = (
                lax.shift_right_logical(tidx[pl.ds(off + 16 * v, 16)], 1))
        cpt = pltpu.async_copy(ent_hbm.at[gidx], tbuf, sem)
        cpt.wait()
        for v in range(CHUNK // 16):
            gidx[pl.ds(16 * v, 16)] = (
                lax.shift_right_logical(ridx[pl.ds(off + 16 * v, 16)], 1))
        cpr = pltpu.async_copy(relemb_hbm.at[gidx], rbuf, sem)
        cpr.wait()

        for g in range(GPC):
            goff = off + g * 16
            slot = g * 16 + iota16
            hsel = (hidx[pl.ds(goff, 16)] & 1) * 64
            tsel = (tidx[pl.ds(goff, 16)] & 1) * 64
            rsel = (ridx[pl.ds(goff, 16)] & 1) * 64
            acc = jnp.zeros((16,), jnp.float32)
            for d in range(DIM):
                rot = (jnp.full((16,), d, jnp.int32) + iota16) & 63
                h = plsc.load_gather(hbuf, [slot, hsel + rot])
                t = plsc.load_gather(tbuf, [slot, tsel + rot])
                r = plsc.load_gather(rbuf, [slot, rsel + rot])
                acc = acc + h * t * r
            score = 1.0 / (1.0 + jnp.exp(-acc))
            oscr[pl.ds(goff, 16)] = score
        return carry

    lax.fori_loop(0, NCHUNK, chunk_body, 0)

    pltpu.sync_copy(oscr, out_hbm.at[pl.ds(base, ROWS_PER_W)])


@functools.partial(
    pl.kernel,
    mesh=_MESH,
    out_type=jax.ShapeDtypeStruct((BATCH,), jnp.float32),
    compiler_params=_PARAMS,
    scratch_types=[
        pltpu.VMEM((ROWS_PER_W,), jnp.int32),
        pltpu.VMEM((ROWS_PER_W,), jnp.int32),
        pltpu.VMEM((ROWS_PER_W,), jnp.int32),
        pltpu.VMEM((CHUNK,), jnp.int32),
        pltpu.VMEM((CHUNK, 2 * DIM), jnp.float32),
        pltpu.VMEM((CHUNK, 2 * DIM), jnp.float32),
        pltpu.VMEM((CHUNK, 2 * DIM), jnp.float32),
        pltpu.VMEM((ROWS_PER_W,), jnp.float32),
        pltpu.SemaphoreType.DMA,
    ],
)
def _gather_sc(*args):
    _gather_body(*args)


def kernel(head, tail, relation, entity_embed, relation_embed):
    ent2 = _transpose_sc(entity_embed.T, entity_embed[TAIL0:].T)
    rel2 = relation_embed.reshape(relation_embed.shape[0] // 2, 2 * DIM)
    return _gather_sc(head.astype(jnp.int32), tail.astype(jnp.int32),
                      relation.astype(jnp.int32), ent2, rel2)


# speedup vs baseline: 3.5171x; 1.9651x over previous
import functools

import jax
import jax.numpy as jnp
from jax import lax
from jax.experimental import pallas as pl
from jax.experimental.pallas import tpu as pltpu
from jax.experimental.pallas import tpu_sc as plsc

BATCH = 16384
DIM = 64
NC = 2
NS = 16
NW = NC * NS
ROWS_PER_W = BATCH // NW
CHUNK = 128
NCHUNK = ROWS_PER_W // CHUNK
GPC = CHUNK // 16

ENT = 1000000
EB = 256
NBLK = 999936 // EB
TAIL0 = 999936
SLOTS = 63

_MESH = plsc.VectorSubcoreMesh(core_axis_name="c", subcore_axis_name="s")
_PARAMS = pltpu.CompilerParams(needs_layout_passes=False,
                               disable_bounds_checks=True)


def _transpose_body(ent_t, ent_tail, out_hbm, abuf0, abuf1, tbuf0, tbuf1,
                    tailbuf, sem_in, sem_out):
    c = lax.axis_index("c")
    s = lax.axis_index("s")
    wid = s * NC + c
    nblk = jnp.where(wid < NBLK - 32 * (NBLK // NW), NBLK // NW + 1,
                     NBLK // NW)
    abufs = (abuf0, abuf1)
    tbufs = (tbuf0, tbuf1)
    iota16 = lax.iota(jnp.int32, 16)

    def e0_of(gi):
        return pl.multiple_of((wid + NW * gi) * EB, EB)

    def start_in(gi, buf):
        pltpu.async_copy(ent_t.at[:, pl.ds(e0_of(gi), EB)],
                         buf.at[:, pl.ds(0, EB)], sem_in)

    start_in(0, abuf0)
    start_in(1, abuf1)

    def slot_pair(ii, carry):
        for b in range(2):
            gi = 2 * ii + b
            ab, tb = abufs[b], tbufs[b]

            @pl.when(gi < nblk)
            def _work():
                pltpu.make_async_copy(ent_t.at[:, pl.ds(0, EB)],
                                      ab.at[:, pl.ds(0, EB)], sem_in).wait()

            @pl.when(jnp.logical_and(gi >= 2, gi - 2 < nblk))
            def _drain():
                pltpu.make_async_copy(tb, out_hbm.at[pl.ds(0, EB // 2)],
                                      sem_out).wait()

            @pl.when(gi < nblk)
            def _transpose():
                @plsc.parallel_loop(0, EB, step=16, unroll=2)
                def _ploop(x0):
                    xv = x0 + iota16
                    pv = lax.shift_right_logical(xv, 1)
                    sel = (xv & 1) * 64
                    for r in range(2):
                        rot = (iota16 + r) & 15
                        for d0 in range(0, DIM, 16):
                            dv = d0 + rot
                            v = plsc.load_gather(ab, [dv, xv])
                            plsc.store_scatter(tb, [pv, sel + dv], v)
                pltpu.async_copy(
                    tb, out_hbm.at[pl.ds(
                        pl.multiple_of((wid + NW * gi) * (EB // 2), EB // 2),
                        EB // 2)],
                    sem_out)

            @pl.when(gi + 2 < nblk)
            def _next_in():
                start_in(gi + 2, ab)

        return carry

    lax.fori_loop(0, SLOTS, slot_pair, 0)

    @pl.when(wid == 2)
    def _tail():
        pltpu.sync_copy(ent_tail, tailbuf)
        for p in range(32):
            ev0 = jnp.full((16,), 2 * p, jnp.int32)
            ev1 = jnp.full((16,), 2 * p + 1, jnp.int32)
            for k in range(8):
                dvec = 16 * (k % 4) + iota16
                ev = ev1 if k >= 4 else ev0
                tbuf0[p, pl.ds(16 * k, 16)] = plsc.load_gather(tailbuf,
                                                               [dvec, ev])
        pltpu.sync_copy(tbuf0.at[pl.ds(0, 32)],
                        out_hbm.at[pl.ds(TAIL0 // 2, 32)])


@functools.partial(
    pl.kernel,
    mesh=_MESH,
    out_type=jax.ShapeDtypeStruct((ENT // 2, 2 * DIM), jnp.float32),
    compiler_params=_PARAMS,
    scratch_types=[
        pltpu.VMEM((DIM, EB + 1), jnp.float32),
        pltpu.VMEM((DIM, EB + 1), jnp.float32),
        pltpu.VMEM((EB // 2, 2 * DIM), jnp.float32),
        pltpu.VMEM((EB // 2, 2 * DIM), jnp.float32),
        pltpu.VMEM((DIM, 64), jnp.float32),
        pltpu.SemaphoreType.DMA,
        pltpu.SemaphoreType.DMA,
    ],
)
def _transpose_sc(*args):
    _transpose_body(*args)


def _gather_body(head_hbm, tail_hbm, rel_hbm, ent_hbm, relemb_hbm, out_hbm,
                 hidx, tidx, ridx, gidx, hbuf, tbuf, rbuf, oscr, sem):
    c = lax.axis_index("c")
    s = lax.axis_index("s")
    wid = s * NC + c
    base = wid * ROWS_PER_W

    pltpu.sync_copy(head_hbm.at[pl.ds(base, ROWS_PER_W)], hidx)
    pltpu.sync_copy(tail_hbm.at[pl.ds(base, ROWS_PER_W)], tidx)
    pltpu.sync_copy(rel_hbm.at[pl.ds(base, ROWS_PER_W)], ridx)

    iota16 = lax.iota(jnp.int32, 16)

    def chunk_body(ck, carry):
        off = pl.multiple_of(ck * CHUNK, CHUNK)
        for v in range(CHUNK // 16):
            gidx[pl.ds(16 * v, 16)] = (
                lax.shift_right_logical(hidx[pl.ds(off + 16 * v, 16)], 1))
        cph = pltpu.async_copy(ent_hbm.at[gidx], hbuf, sem)
        cph.wait()
        for v in range(CHUNK // 16):
            gidx[pl.ds(16 * v, 16)] = (
                lax.shift_right_logical(tidx[pl.ds(off + 16 * v, 16)], 1))
        cpt = pltpu.async_copy(ent_hbm.at[gidx], tbuf, sem)
        cpt.wait()
        for v in range(CHUNK // 16):
            gidx[pl.ds(16 * v, 16)] = (
                lax.shift_right_logical(ridx[pl.ds(off + 16 * v, 16)], 1))
        cpr = pltpu.async_copy(relemb_hbm.at[gidx], rbuf, sem)
        cpr.wait()

        for g in range(GPC):
            goff = off + g * 16
            slot = g * 16 + iota16
            hsel = (hidx[pl.ds(goff, 16)] & 1) * 64
            tsel = (tidx[pl.ds(goff, 16)] & 1) * 64
            rsel = (ridx[pl.ds(goff, 16)] & 1) * 64
            acc = jnp.zeros((16,), jnp.float32)
            for d in range(DIM):
                rot = (jnp.full((16,), d, jnp.int32) + iota16) & 63
                h = plsc.load_gather(hbuf, [slot, hsel + rot])
                t = plsc.load_gather(tbuf, [slot, tsel + rot])
                r = plsc.load_gather(rbuf, [slot, rsel + rot])
                acc = acc + h * t * r
            score = 1.0 / (1.0 + jnp.exp(-acc))
            oscr[pl.ds(goff, 16)] = score
        return carry

    lax.fori_loop(0, NCHUNK, chunk_body, 0)

    pltpu.sync_copy(oscr, out_hbm.at[pl.ds(base, ROWS_PER_W)])


@functools.partial(
    pl.kernel,
    mesh=_MESH,
    out_type=jax.ShapeDtypeStruct((BATCH,), jnp.float32),
    compiler_params=_PARAMS,
    scratch_types=[
        pltpu.VMEM((ROWS_PER_W,), jnp.int32),
        pltpu.VMEM((ROWS_PER_W,), jnp.int32),
        pltpu.VMEM((ROWS_PER_W,), jnp.int32),
        pltpu.VMEM((CHUNK,), jnp.int32),
        pltpu.VMEM((CHUNK, 2 * DIM), jnp.float32),
        pltpu.VMEM((CHUNK, 2 * DIM), jnp.float32),
        pltpu.VMEM((CHUNK, 2 * DIM), jnp.float32),
        pltpu.VMEM((ROWS_PER_W,), jnp.float32),
        pltpu.SemaphoreType.DMA,
    ],
)
def _gather_sc(*args):
    _gather_body(*args)


def kernel(head, tail, relation, entity_embed, relation_embed):
    ent2 = _transpose_sc(entity_embed.T, entity_embed[TAIL0:].T)
    rel2 = relation_embed.reshape(relation_embed.shape[0] // 2, 2 * DIM)
    return _gather_sc(head.astype(jnp.int32), tail.astype(jnp.int32),
                      relation.astype(jnp.int32), ent2, rel2)


# speedup vs baseline: 3.7610x; 1.0693x over previous
import functools

import jax
import jax.numpy as jnp
from jax import lax
from jax.experimental import pallas as pl
from jax.experimental.pallas import tpu as pltpu
from jax.experimental.pallas import tpu_sc as plsc

BATCH = 16384
DIM = 64
NC = 2
NS = 16
NW = NC * NS
ROWS_PER_W = BATCH // NW
CHUNK = 128
NCHUNK = ROWS_PER_W // CHUNK
GPC = CHUNK // 16

ENT = 1000000
EB = 384
NBLK = 999936 // EB
TAIL0 = 999936
SLOTS = 43

_MESH = plsc.VectorSubcoreMesh(core_axis_name="c", subcore_axis_name="s")
_PARAMS = pltpu.CompilerParams(needs_layout_passes=False,
                               disable_bounds_checks=True)


def _transpose_body(ent_t, ent_tail, out_hbm, abuf0, abuf1, tbuf0, tbuf1,
                    tailbuf, sem_in, sem_out):
    c = lax.axis_index("c")
    s = lax.axis_index("s")
    wid = s * NC + c
    nblk = jnp.where(wid < NBLK - 32 * (NBLK // NW), NBLK // NW + 1,
                     NBLK // NW)
    abufs = (abuf0, abuf1)
    tbufs = (tbuf0, tbuf1)
    iota16 = lax.iota(jnp.int32, 16)

    def e0_of(gi):
        return pl.multiple_of((wid + NW * gi) * EB, EB)

    def start_in(gi, buf):
        pltpu.async_copy(ent_t.at[:, pl.ds(e0_of(gi), EB)],
                         buf.at[:, pl.ds(0, EB)], sem_in)

    start_in(0, abuf0)
    start_in(1, abuf1)

    def slot_pair(ii, carry):
        for b in range(2):
            gi = 2 * ii + b
            ab, tb = abufs[b], tbufs[b]

            @pl.when(gi < nblk)
            def _work():
                pltpu.make_async_copy(ent_t.at[:, pl.ds(0, EB)],
                                      ab.at[:, pl.ds(0, EB)], sem_in).wait()

            @pl.when(jnp.logical_and(gi >= 2, gi - 2 < nblk))
            def _drain():
                pltpu.make_async_copy(tb, out_hbm.at[pl.ds(0, EB // 2)],
                                      sem_out).wait()

            @pl.when(gi < nblk)
            def _transpose():
                @plsc.parallel_loop(0, EB, step=16, unroll=4)
                def _ploop(x0):
                    xv = x0 + iota16
                    pv = lax.shift_right_logical(xv, 1)
                    sel = (xv & 1) * 64
                    for r in range(16):
                        rot = (iota16 + r) & 15
                        for d0 in range(0, DIM, 16):
                            dv = d0 + rot
                            v = plsc.load_gather(ab, [dv, xv])
                            plsc.store_scatter(tb, [pv, sel + dv], v)
                pltpu.async_copy(
                    tb, out_hbm.at[pl.ds(
                        pl.multiple_of((wid + NW * gi) * (EB // 2), EB // 2),
                        EB // 2)],
                    sem_out)

            @pl.when(gi + 2 < nblk)
            def _next_in():
                start_in(gi + 2, ab)

        return carry

    lax.fori_loop(0, SLOTS, slot_pair, 0)

    @pl.when(wid == 2)
    def _tail():
        pltpu.sync_copy(ent_tail, tailbuf)
        for p in range(32):
            ev0 = jnp.full((16,), 2 * p, jnp.int32)
            ev1 = jnp.full((16,), 2 * p + 1, jnp.int32)
            for k in range(8):
                dvec = 16 * (k % 4) + iota16
                ev = ev1 if k >= 4 else ev0
                tbuf0[p, pl.ds(16 * k, 16)] = plsc.load_gather(tailbuf,
                                                               [dvec, ev])
        pltpu.sync_copy(tbuf0.at[pl.ds(0, 32)],
                        out_hbm.at[pl.ds(TAIL0 // 2, 32)])


@functools.partial(
    pl.kernel,
    mesh=_MESH,
    out_type=jax.ShapeDtypeStruct((ENT // 2, 2 * DIM), jnp.float32),
    compiler_params=_PARAMS,
    scratch_types=[
        pltpu.VMEM((DIM, EB + 1), jnp.float32),
        pltpu.VMEM((DIM, EB + 1), jnp.float32),
        pltpu.VMEM((EB // 2, 2 * DIM), jnp.float32),
        pltpu.VMEM((EB // 2, 2 * DIM), jnp.float32),
        pltpu.VMEM((DIM, 64), jnp.float32),
        pltpu.SemaphoreType.DMA,
        pltpu.SemaphoreType.DMA,
    ],
)
def _transpose_sc(*args):
    _transpose_body(*args)


def _gather_body(head_hbm, tail_hbm, rel_hbm, ent_hbm, relemb_hbm, out_hbm,
                 hidx, tidx, ridx, gxh, gxt, gxr, hbuf0, hbuf1, tbuf0, tbuf1,
                 rbuf0, rbuf1, oscr, sem):
    c = lax.axis_index("c")
    s = lax.axis_index("s")
    wid = s * NC + c
    base = wid * ROWS_PER_W

    pltpu.sync_copy(head_hbm.at[pl.ds(base, ROWS_PER_W)], hidx)
    pltpu.sync_copy(tail_hbm.at[pl.ds(base, ROWS_PER_W)], tidx)
    pltpu.sync_copy(rel_hbm.at[pl.ds(base, ROWS_PER_W)], ridx)

    iota16 = lax.iota(jnp.int32, 16)
    hbufs = (hbuf0, hbuf1)
    tbufs = (tbuf0, tbuf1)
    rbufs = (rbuf0, rbuf1)

    for v in range(ROWS_PER_W // 16):
        sl = pl.ds(16 * v, 16)
        gxh[sl] = lax.shift_right_logical(hidx[sl], 1)
        gxt[sl] = lax.shift_right_logical(tidx[sl], 1)
        gxr[sl] = lax.shift_right_logical(ridx[sl], 1)

    def fire(ck, b):
        off = pl.ds(ck * CHUNK, CHUNK)
        pltpu.async_copy(ent_hbm.at[gxh.at[off]], hbufs[b], sem)
        pltpu.async_copy(ent_hbm.at[gxt.at[off]], tbufs[b], sem)
        pltpu.async_copy(relemb_hbm.at[gxr.at[off]], rbufs[b], sem)

    def drain(b):
        pltpu.make_async_copy(ent_hbm.at[gxh.at[pl.ds(0, CHUNK)]], hbufs[b],
                              sem).wait()
        pltpu.make_async_copy(ent_hbm.at[gxt.at[pl.ds(0, CHUNK)]], tbufs[b],
                              sem).wait()
        pltpu.make_async_copy(relemb_hbm.at[gxr.at[pl.ds(0, CHUNK)]],
                              rbufs[b], sem).wait()

    fire(0, 0)
    for ck in range(NCHUNK):
        b = ck & 1
        if ck + 1 < NCHUNK:
            fire(ck + 1, 1 - b)
        drain(b)
        hb, tb, rb = hbufs[b], tbufs[b], rbufs[b]
        off = ck * CHUNK

        def group_body(g, carry):
            goff = pl.multiple_of(off + g * 16, 16)
            slot = g * 16 + iota16
            hsel = (hidx[pl.ds(goff, 16)] & 1) * 64
            tsel = (tidx[pl.ds(goff, 16)] & 1) * 64
            rsel = (ridx[pl.ds(goff, 16)] & 1) * 64
            acc = jnp.zeros((16,), jnp.float32)
            for d in range(DIM):
                rot = (jnp.full((16,), d, jnp.int32) + iota16) & 63
                h = plsc.load_gather(hb, [slot, hsel + rot])
                t = plsc.load_gather(tb, [slot, tsel + rot])
                r = plsc.load_gather(rb, [slot, rsel + rot])
                acc = acc + h * t * r
            score = 1.0 / (1.0 + jnp.exp(-acc))
            oscr[pl.ds(goff, 16)] = score
            return carry

        lax.fori_loop(0, GPC, group_body, 0)

    pltpu.sync_copy(oscr, out_hbm.at[pl.ds(base, ROWS_PER_W)])


@functools.partial(
    pl.kernel,
    mesh=_MESH,
    out_type=jax.ShapeDtypeStruct((BATCH,), jnp.float32),
    compiler_params=_PARAMS,
    scratch_types=[
        pltpu.VMEM((ROWS_PER_W,), jnp.int32),
        pltpu.VMEM((ROWS_PER_W,), jnp.int32),
        pltpu.VMEM((ROWS_PER_W,), jnp.int32),
        pltpu.VMEM((ROWS_PER_W,), jnp.int32),
        pltpu.VMEM((ROWS_PER_W,), jnp.int32),
        pltpu.VMEM((ROWS_PER_W,), jnp.int32),
        pltpu.VMEM((CHUNK, 2 * DIM), jnp.float32),
        pltpu.VMEM((CHUNK, 2 * DIM), jnp.float32),
        pltpu.VMEM((CHUNK, 2 * DIM), jnp.float32),
        pltpu.VMEM((CHUNK, 2 * DIM), jnp.float32),
        pltpu.VMEM((CHUNK, 2 * DIM), jnp.float32),
        pltpu.VMEM((CHUNK, 2 * DIM), jnp.float32),
        pltpu.VMEM((ROWS_PER_W,), jnp.float32),
        pltpu.SemaphoreType.DMA,
    ],
)
def _gather_sc(*args):
    _gather_body(*args)


def kernel(head, tail, relation, entity_embed, relation_embed):
    ent2 = _transpose_sc(entity_embed.T, entity_embed[TAIL0:].T)
    rel2 = relation_embed.reshape(relation_embed.shape[0] // 2, 2 * DIM)
    return _gather_sc(head.astype(jnp.int32), tail.astype(jnp.int32),
                      relation.astype(jnp.int32), ent2, rel2)


# speedup vs baseline: 3.7710x; 1.0027x over previous
import functools

import jax
import jax.numpy as jnp
from jax import lax
from jax.experimental import pallas as pl
from jax.experimental.pallas import tpu as pltpu
from jax.experimental.pallas import tpu_sc as plsc

BATCH = 16384
DIM = 64
NC = 2
NS = 16
NW = NC * NS
ROWS_PER_W = BATCH // NW
CHUNK = 128
NCHUNK = ROWS_PER_W // CHUNK
GPC = CHUNK // 16

ENT = 1000000
EB = 384
NBLK = 999936 // EB
TAIL0 = 999936
SLOTS = 43

_MESH = plsc.VectorSubcoreMesh(core_axis_name="c", subcore_axis_name="s")
_PARAMS = pltpu.CompilerParams(needs_layout_passes=False,
                               disable_bounds_checks=True)


def _transpose_body(ent_t, ent_tail, out_hbm, abuf0, abuf1, tbuf0, tbuf1,
                    tailbuf, sem_in, sem_out):
    c = lax.axis_index("c")
    s = lax.axis_index("s")
    wid = s * NC + c
    nblk = jnp.where(wid < NBLK - 32 * (NBLK // NW), NBLK // NW + 1,
                     NBLK // NW)
    abufs = (abuf0, abuf1)
    tbufs = (tbuf0, tbuf1)
    iota16 = lax.iota(jnp.int32, 16)

    def e0_of(gi):
        return pl.multiple_of((wid + NW * gi) * EB, EB)

    def start_in(gi, buf):
        pltpu.async_copy(ent_t.at[:, pl.ds(e0_of(gi), EB)], buf, sem_in)

    start_in(0, abuf0)
    start_in(1, abuf1)

    def slot_pair(ii, carry):
        for b in range(2):
            gi = 2 * ii + b
            ab, tb = abufs[b], tbufs[b]

            @pl.when(gi < nblk)
            def _work():
                pltpu.make_async_copy(ent_t.at[:, pl.ds(0, EB)], ab,
                                      sem_in).wait()

            @pl.when(jnp.logical_and(gi >= 2, gi - 2 < nblk))
            def _drain():
                pltpu.make_async_copy(tb, out_hbm.at[pl.ds(0, EB // 2)],
                                      sem_out).wait()

            @pl.when(gi < nblk)
            def _transpose():
                @plsc.parallel_loop(0, EB, step=16, unroll=4)
                def _ploop(x0):
                    xv = x0 + iota16
                    pv = lax.shift_right_logical(xv, 1)
                    sel = (xv & 1) * 64
                    for r in range(16):
                        rot = (iota16 + r) & 15
                        for d0 in range(0, DIM, 16):
                            dv = d0 + rot
                            v = plsc.load_gather(ab, [dv, xv])
                            plsc.store_scatter(tb, [pv, sel + dv], v)
                pltpu.async_copy(
                    tb, out_hbm.at[pl.ds(
                        pl.multiple_of((wid + NW * gi) * (EB // 2), EB // 2),
                        EB // 2)],
                    sem_out)

            @pl.when(gi + 2 < nblk)
            def _next_in():
                start_in(gi + 2, ab)

        return carry

    lax.fori_loop(0, SLOTS, slot_pair, 0)

    @pl.when(wid == 2)
    def _tail():
        pltpu.sync_copy(ent_tail, tailbuf)
        for p in range(32):
            ev0 = jnp.full((16,), 2 * p, jnp.int32)
            ev1 = jnp.full((16,), 2 * p + 1, jnp.int32)
            for k in range(8):
                dvec = 16 * (k % 4) + iota16
                ev = ev1 if k >= 4 else ev0
                tbuf0[p, pl.ds(16 * k, 16)] = plsc.load_gather(tailbuf,
                                                               [dvec, ev])
        pltpu.sync_copy(tbuf0.at[pl.ds(0, 32)],
                        out_hbm.at[pl.ds(TAIL0 // 2, 32)])


@functools.partial(
    pl.kernel,
    mesh=_MESH,
    out_type=jax.ShapeDtypeStruct((ENT // 2, 2 * DIM), jnp.float32),
    compiler_params=_PARAMS,
    scratch_types=[
        pltpu.VMEM((DIM, EB), jnp.float32),
        pltpu.VMEM((DIM, EB), jnp.float32),
        pltpu.VMEM((EB // 2, 2 * DIM), jnp.float32),
        pltpu.VMEM((EB // 2, 2 * DIM), jnp.float32),
        pltpu.VMEM((DIM, 64), jnp.float32),
        pltpu.SemaphoreType.DMA,
        pltpu.SemaphoreType.DMA,
    ],
)
def _transpose_sc(*args):
    _transpose_body(*args)


def _gather_body(head_hbm, tail_hbm, rel_hbm, ent_hbm, relemb_hbm, out_hbm,
                 hidx, tidx, ridx, gxh, gxt, gxr, hbuf0, hbuf1, tbuf0, tbuf1,
                 rbuf0, rbuf1, oscr, sem):
    c = lax.axis_index("c")
    s = lax.axis_index("s")
    wid = s * NC + c
    base = wid * ROWS_PER_W

    pltpu.sync_copy(head_hbm.at[pl.ds(base, ROWS_PER_W)], hidx)
    pltpu.sync_copy(tail_hbm.at[pl.ds(base, ROWS_PER_W)], tidx)
    pltpu.sync_copy(rel_hbm.at[pl.ds(base, ROWS_PER_W)], ridx)

    iota16 = lax.iota(jnp.int32, 16)
    hbufs = (hbuf0, hbuf1)
    tbufs = (tbuf0, tbuf1)
    rbufs = (rbuf0, rbuf1)

    for v in range(ROWS_PER_W // 16):
        sl = pl.ds(16 * v, 16)
        gxh[sl] = lax.shift_right_logical(hidx[sl], 1)
        gxt[sl] = lax.shift_right_logical(tidx[sl], 1)
        gxr[sl] = lax.shift_right_logical(ridx[sl], 1)

    def fire(ck, b):
        off = pl.ds(ck * CHUNK, CHUNK)
        pltpu.async_copy(ent_hbm.at[gxh.at[off]], hbufs[b], sem)
        pltpu.async_copy(ent_hbm.at[gxt.at[off]], tbufs[b], sem)
        pltpu.async_copy(relemb_hbm.at[gxr.at[off]], rbufs[b], sem)

    def drain(b):
        pltpu.make_async_copy(ent_hbm.at[gxh.at[pl.ds(0, CHUNK)]], hbufs[b],
                              sem).wait()
        pltpu.make_async_copy(ent_hbm.at[gxt.at[pl.ds(0, CHUNK)]], tbufs[b],
                              sem).wait()
        pltpu.make_async_copy(relemb_hbm.at[gxr.at[pl.ds(0, CHUNK)]],
                              rbufs[b], sem).wait()

    fire(0, 0)
    for ck in range(NCHUNK):
        b = ck & 1
        if ck + 1 < NCHUNK:
            fire(ck + 1, 1 - b)
        drain(b)
        hb, tb, rb = hbufs[b], tbufs[b], rbufs[b]
        off = ck * CHUNK

        def group_body(g, carry):
            goff = pl.multiple_of(off + g * 16, 16)
            slot = g * 16 + iota16
            hsel = (hidx[pl.ds(goff, 16)] & 1) * 64
            tsel = (tidx[pl.ds(goff, 16)] & 1) * 64
            rsel = (ridx[pl.ds(goff, 16)] & 1) * 64
            acc = jnp.zeros((16,), jnp.float32)
            for d in range(DIM):
                rot = (jnp.full((16,), d, jnp.int32) + iota16) & 63
                h = plsc.load_gather(hb, [slot, hsel + rot])
                t = plsc.load_gather(tb, [slot, tsel + rot])
                r = plsc.load_gather(rb, [slot, rsel + rot])
                acc = acc + h * t * r
            score = 1.0 / (1.0 + jnp.exp(-acc))
            oscr[pl.ds(goff, 16)] = score
            return carry

        lax.fori_loop(0, GPC, group_body, 0)

    pltpu.sync_copy(oscr, out_hbm.at[pl.ds(base, ROWS_PER_W)])


@functools.partial(
    pl.kernel,
    mesh=_MESH,
    out_type=jax.ShapeDtypeStruct((BATCH,), jnp.float32),
    compiler_params=_PARAMS,
    scratch_types=[
        pltpu.VMEM((ROWS_PER_W,), jnp.int32),
        pltpu.VMEM((ROWS_PER_W,), jnp.int32),
        pltpu.VMEM((ROWS_PER_W,), jnp.int32),
        pltpu.VMEM((ROWS_PER_W,), jnp.int32),
        pltpu.VMEM((ROWS_PER_W,), jnp.int32),
        pltpu.VMEM((ROWS_PER_W,), jnp.int32),
        pltpu.VMEM((CHUNK, 2 * DIM), jnp.float32),
        pltpu.VMEM((CHUNK, 2 * DIM), jnp.float32),
        pltpu.VMEM((CHUNK, 2 * DIM), jnp.float32),
        pltpu.VMEM((CHUNK, 2 * DIM), jnp.float32),
        pltpu.VMEM((CHUNK, 2 * DIM), jnp.float32),
        pltpu.VMEM((CHUNK, 2 * DIM), jnp.float32),
        pltpu.VMEM((ROWS_PER_W,), jnp.float32),
        pltpu.SemaphoreType.DMA,
    ],
)
def _gather_sc(*args):
    _gather_body(*args)


def kernel(head, tail, relation, entity_embed, relation_embed):
    ent2 = _transpose_sc(entity_embed.T, entity_embed[TAIL0:].T)
    rel2 = relation_embed.reshape(relation_embed.shape[0] // 2, 2 * DIM)
    return _gather_sc(head.astype(jnp.int32), tail.astype(jnp.int32),
                      relation.astype(jnp.int32), ent2, rel2)


# speedup vs baseline: 3.7770x; 1.0016x over previous
import functools

import jax
import jax.numpy as jnp
from jax import lax
from jax.experimental import pallas as pl
from jax.experimental.pallas import tpu as pltpu
from jax.experimental.pallas import tpu_sc as plsc

BATCH = 16384
DIM = 64
NC = 2
NS = 16
NW = NC * NS
ROWS_PER_W = BATCH // NW
CHUNK = 128
NCHUNK = ROWS_PER_W // CHUNK
GPC = CHUNK // 16

ENT = 1000000
EB = 384
NBLK = 999936 // EB
TAIL0 = 999936
SLOTS = 43

_MESH = plsc.VectorSubcoreMesh(core_axis_name="c", subcore_axis_name="s")
_PARAMS = pltpu.CompilerParams(needs_layout_passes=False,
                               disable_bounds_checks=True,
                               skip_device_barrier=True)


def _transpose_body(ent_t, ent_tail, out_hbm, abuf0, abuf1, tbuf0, tbuf1,
                    tailbuf, sem_in, sem_out):
    c = lax.axis_index("c")
    s = lax.axis_index("s")
    wid = s * NC + c
    nblk = jnp.where(wid < NBLK - 32 * (NBLK // NW), NBLK // NW + 1,
                     NBLK // NW)
    abufs = (abuf0, abuf1)
    tbufs = (tbuf0, tbuf1)
    iota16 = lax.iota(jnp.int32, 16)

    def e0_of(gi):
        return pl.multiple_of((wid + NW * gi) * EB, EB)

    def start_in(gi, buf):
        pltpu.async_copy(ent_t.at[:, pl.ds(e0_of(gi), EB)], buf, sem_in)

    start_in(0, abuf0)
    start_in(1, abuf1)

    def slot_pair(ii, carry):
        for b in range(2):
            gi = 2 * ii + b
            ab, tb = abufs[b], tbufs[b]

            @pl.when(gi < nblk)
            def _work():
                pltpu.make_async_copy(ent_t.at[:, pl.ds(0, EB)], ab,
                                      sem_in).wait()

            @pl.when(jnp.logical_and(gi >= 2, gi - 2 < nblk))
            def _drain():
                pltpu.make_async_copy(tb, out_hbm.at[pl.ds(0, EB // 2)],
                                      sem_out).wait()

            @pl.when(gi < nblk)
            def _transpose():
                @plsc.parallel_loop(0, EB, step=16, unroll=4)
                def _ploop(x0):
                    xv = x0 + iota16
                    pv = lax.shift_right_logical(xv, 1)
                    sel = (xv & 1) * 64
                    for r in range(16):
                        rot = (iota16 + r) & 15
                        for d0 in range(0, DIM, 16):
                            dv = d0 + rot
                            v = plsc.load_gather(ab, [dv, xv])
                            plsc.store_scatter(tb, [pv, sel + dv], v)
                pltpu.async_copy(
                    tb, out_hbm.at[pl.ds(
                        pl.multiple_of((wid + NW * gi) * (EB // 2), EB // 2),
                        EB // 2)],
                    sem_out)

            @pl.when(gi + 2 < nblk)
            def _next_in():
                start_in(gi + 2, ab)

        return carry

    lax.fori_loop(0, SLOTS, slot_pair, 0)

    @pl.when(wid == 2)
    def _tail():
        pltpu.sync_copy(ent_tail, tailbuf)
        for p in range(32):
            ev0 = jnp.full((16,), 2 * p, jnp.int32)
            ev1 = jnp.full((16,), 2 * p + 1, jnp.int32)
            for k in range(8):
                dvec = 16 * (k % 4) + iota16
                ev = ev1 if k >= 4 else ev0
                tbuf0[p, pl.ds(16 * k, 16)] = plsc.load_gather(tailbuf,
                                                               [dvec, ev])
        pltpu.sync_copy(tbuf0.at[pl.ds(0, 32)],
                        out_hbm.at[pl.ds(TAIL0 // 2, 32)])


@functools.partial(
    pl.kernel,
    mesh=_MESH,
    out_type=jax.ShapeDtypeStruct((ENT // 2, 2 * DIM), jnp.float32),
    compiler_params=_PARAMS,
    scratch_types=[
        pltpu.VMEM((DIM, EB), jnp.float32),
        pltpu.VMEM((DIM, EB), jnp.float32),
        pltpu.VMEM((EB // 2, 2 * DIM), jnp.float32),
        pltpu.VMEM((EB // 2, 2 * DIM), jnp.float32),
        pltpu.VMEM((DIM, 64), jnp.float32),
        pltpu.SemaphoreType.DMA,
        pltpu.SemaphoreType.DMA,
    ],
)
def _transpose_sc(*args):
    _transpose_body(*args)


def _gather_body(head_hbm, tail_hbm, rel_hbm, ent_hbm, relemb_hbm, out_hbm,
                 hidx, tidx, ridx, gxh, gxt, gxr, hbuf0, hbuf1, tbuf0, tbuf1,
                 rbuf0, rbuf1, oscr, sem):
    c = lax.axis_index("c")
    s = lax.axis_index("s")
    wid = s * NC + c
    base = wid * ROWS_PER_W

    pltpu.sync_copy(head_hbm.at[pl.ds(base, ROWS_PER_W)], hidx)
    pltpu.sync_copy(tail_hbm.at[pl.ds(base, ROWS_PER_W)], tidx)
    pltpu.sync_copy(rel_hbm.at[pl.ds(base, ROWS_PER_W)], ridx)

    iota16 = lax.iota(jnp.int32, 16)
    hbufs = (hbuf0, hbuf1)
    tbufs = (tbuf0, tbuf1)
    rbufs = (rbuf0, rbuf1)

    for v in range(ROWS_PER_W // 16):
        sl = pl.ds(16 * v, 16)
        gxh[sl] = lax.shift_right_logical(hidx[sl], 1)
        gxt[sl] = lax.shift_right_logical(tidx[sl], 1)
        gxr[sl] = lax.shift_right_logical(ridx[sl], 1)

    def fire(ck, b):
        off = pl.ds(ck * CHUNK, CHUNK)
        pltpu.async_copy(ent_hbm.at[gxh.at[off]], hbufs[b], sem)
        pltpu.async_copy(ent_hbm.at[gxt.at[off]], tbufs[b], sem)
        pltpu.async_copy(relemb_hbm.at[gxr.at[off]], rbufs[b], sem)

    def drain(b):
        pltpu.make_async_copy(ent_hbm.at[gxh.at[pl.ds(0, CHUNK)]], hbufs[b],
                              sem).wait()
        pltpu.make_async_copy(ent_hbm.at[gxt.at[pl.ds(0, CHUNK)]], tbufs[b],
                              sem).wait()
        pltpu.make_async_copy(relemb_hbm.at[gxr.at[pl.ds(0, CHUNK)]],
                              rbufs[b], sem).wait()

    fire(0, 0)
    for ck in range(NCHUNK):
        b = ck & 1
        if ck + 1 < NCHUNK:
            fire(ck + 1, 1 - b)
        drain(b)
        hb, tb, rb = hbufs[b], tbufs[b], rbufs[b]
        off = ck * CHUNK

        def group_body(g, carry):
            goff = pl.multiple_of(off + g * 16, 16)
            slot = g * 16 + iota16
            hsel = (hidx[pl.ds(goff, 16)] & 1) * 64
            tsel = (tidx[pl.ds(goff, 16)] & 1) * 64
            rsel = (ridx[pl.ds(goff, 16)] & 1) * 64
            acc = jnp.zeros((16,), jnp.float32)
            for d in range(DIM):
                rot = (jnp.full((16,), d, jnp.int32) + iota16) & 63
                h = plsc.load_gather(hb, [slot, hsel + rot])
                t = plsc.load_gather(tb, [slot, tsel + rot])
                r = plsc.load_gather(rb, [slot, rsel + rot])
                acc = acc + h * t * r
            score = 1.0 / (1.0 + jnp.exp(-acc))
            oscr[pl.ds(goff, 16)] = score
            return carry

        lax.fori_loop(0, GPC, group_body, 0)

    pltpu.sync_copy(oscr, out_hbm.at[pl.ds(base, ROWS_PER_W)])


@functools.partial(
    pl.kernel,
    mesh=_MESH,
    out_type=jax.ShapeDtypeStruct((BATCH,), jnp.float32),
    compiler_params=_PARAMS,
    scratch_types=[
        pltpu.VMEM((ROWS_PER_W,), jnp.int32),
        pltpu.VMEM((ROWS_PER_W,), jnp.int32),
        pltpu.VMEM((ROWS_PER_W,), jnp.int32),
        pltpu.VMEM((ROWS_PER_W,), jnp.int32),
        pltpu.VMEM((ROWS_PER_W,), jnp.int32),
        pltpu.VMEM((ROWS_PER_W,), jnp.int32),
        pltpu.VMEM((CHUNK, 2 * DIM), jnp.float32),
        pltpu.VMEM((CHUNK, 2 * DIM), jnp.float32),
        pltpu.VMEM((CHUNK, 2 * DIM), jnp.float32),
        pltpu.VMEM((CHUNK, 2 * DIM), jnp.float32),
        pltpu.VMEM((CHUNK, 2 * DIM), jnp.float32),
        pltpu.VMEM((CHUNK, 2 * DIM), jnp.float32),
        pltpu.VMEM((ROWS_PER_W,), jnp.float32),
        pltpu.SemaphoreType.DMA,
    ],
)
def _gather_sc(*args):
    _gather_body(*args)


def kernel(head, tail, relation, entity_embed, relation_embed):
    ent2 = _transpose_sc(entity_embed.T, entity_embed[TAIL0:].T)
    rel2 = relation_embed.reshape(relation_embed.shape[0] // 2, 2 * DIM)
    return _gather_sc(head.astype(jnp.int32), tail.astype(jnp.int32),
                      relation.astype(jnp.int32), ent2, rel2)


# speedup vs baseline: 3.7855x; 1.0023x over previous
import functools

import jax
import jax.numpy as jnp
from jax import lax
from jax.experimental import pallas as pl
from jax.experimental.pallas import tpu as pltpu
from jax.experimental.pallas import tpu_sc as plsc

BATCH = 16384
DIM = 64
NC = 2
NS = 16
NW = NC * NS
ROWS_PER_W = BATCH // NW
CHUNK = 128
NCHUNK = ROWS_PER_W // CHUNK
GPC = CHUNK // 16

ENT = 1000000
EB = 384
NBLK = 999936 // EB
TAIL0 = 999936
SLOTS = 43

_MESH = plsc.VectorSubcoreMesh(core_axis_name="c", subcore_axis_name="s")
_PARAMS = pltpu.CompilerParams(needs_layout_passes=False,
                               disable_bounds_checks=True)


def _transpose_body(ent_t, ent_tail, out_hbm, abuf0, abuf1, tbuf0, tbuf1,
                    tailbuf, sem_in, sem_out):
    c = lax.axis_index("c")
    s = lax.axis_index("s")
    wid = s * NC + c
    nblk = jnp.where(wid < NBLK - 32 * (NBLK // NW), NBLK // NW + 1,
                     NBLK // NW)
    abufs = (abuf0, abuf1)
    tbufs = (tbuf0, tbuf1)
    iota16 = lax.iota(jnp.int32, 16)

    def e0_of(gi):
        return pl.multiple_of((wid + NW * gi) * EB, EB)

    def start_in(gi, buf):
        pltpu.async_copy(ent_t.at[:, pl.ds(e0_of(gi), EB)], buf, sem_in)

    start_in(0, abuf0)
    start_in(1, abuf1)

    def slot_pair(ii, carry):
        for b in range(2):
            gi = 2 * ii + b
            ab, tb = abufs[b], tbufs[b]

            @pl.when(gi < nblk)
            def _work():
                pltpu.make_async_copy(ent_t.at[:, pl.ds(0, EB)], ab,
                                      sem_in).wait()

            @pl.when(jnp.logical_and(gi >= 2, gi - 2 < nblk))
            def _drain():
                pltpu.make_async_copy(tb, out_hbm.at[pl.ds(0, EB // 2)],
                                      sem_out).wait()

            @pl.when(gi < nblk)
            def _transpose():
                @plsc.parallel_loop(0, EB, step=16, unroll=4)
                def _ploop(x0):
                    xv = x0 + iota16
                    pv = lax.shift_right_logical(xv, 1)
                    sel = (xv & 1) * 64
                    for r in range(16):
                        rot = (iota16 + r) & 15
                        for d0 in range(0, DIM, 16):
                            dv = d0 + rot
                            v = plsc.load_gather(ab, [dv, xv])
                            plsc.store_scatter(tb, [pv, sel + dv], v)
                pltpu.async_copy(
                    tb, out_hbm.at[pl.ds(
                        pl.multiple_of((wid + NW * gi) * (EB // 2), EB // 2),
                        EB // 2)],
                    sem_out)

            @pl.when(gi + 2 < nblk)
            def _next_in():
                start_in(gi + 2, ab)

        return carry

    lax.fori_loop(0, SLOTS, slot_pair, 0)

    @pl.when(wid == 2)
    def _tail():
        pltpu.sync_copy(ent_tail, tailbuf)
        for p in range(32):
            ev0 = jnp.full((16,), 2 * p, jnp.int32)
            ev1 = jnp.full((16,), 2 * p + 1, jnp.int32)
            for k in range(8):
                dvec = 16 * (k % 4) + iota16
                ev = ev1 if k >= 4 else ev0
                tbuf0[p, pl.ds(16 * k, 16)] = plsc.load_gather(tailbuf,
                                                               [dvec, ev])
        pltpu.sync_copy(tbuf0.at[pl.ds(0, 32)],
                        out_hbm.at[pl.ds(TAIL0 // 2, 32)])


@functools.partial(
    pl.kernel,
    mesh=_MESH,
    out_type=jax.ShapeDtypeStruct((ENT // 2, 2 * DIM), jnp.float32),
    compiler_params=_PARAMS,
    scratch_types=[
        pltpu.VMEM((DIM, EB), jnp.float32),
        pltpu.VMEM((DIM, EB), jnp.float32),
        pltpu.VMEM((EB // 2, 2 * DIM), jnp.float32),
        pltpu.VMEM((EB // 2, 2 * DIM), jnp.float32),
        pltpu.VMEM((DIM, 64), jnp.float32),
        pltpu.SemaphoreType.DMA,
        pltpu.SemaphoreType.DMA,
    ],
)
def _transpose_sc(*args):
    _transpose_body(*args)


def _gather_body(head_hbm, tail_hbm, rel_hbm, ent_hbm, relemb_hbm, out_hbm,
                 hidx, tidx, ridx, gxh, gxt, gxr, hbuf0, hbuf1, tbuf0, tbuf1,
                 rbuf0, rbuf1, oscr, sem):
    c = lax.axis_index("c")
    s = lax.axis_index("s")
    wid = s * NC + c
    base = wid * ROWS_PER_W

    pltpu.sync_copy(head_hbm.at[pl.ds(base, ROWS_PER_W)], hidx)
    pltpu.sync_copy(tail_hbm.at[pl.ds(base, ROWS_PER_W)], tidx)
    pltpu.sync_copy(rel_hbm.at[pl.ds(base, ROWS_PER_W)], ridx)

    iota16 = lax.iota(jnp.int32, 16)
    hbufs = (hbuf0, hbuf1)
    tbufs = (tbuf0, tbuf1)
    rbufs = (rbuf0, rbuf1)

    for v in range(ROWS_PER_W // 16):
        sl = pl.ds(16 * v, 16)
        gxh[sl] = lax.shift_right_logical(hidx[sl], 1)
        gxt[sl] = lax.shift_right_logical(tidx[sl], 1)
        gxr[sl] = lax.shift_right_logical(ridx[sl], 1)

    def fire(ck, b):
        off = pl.ds(ck * CHUNK, CHUNK)
        pltpu.async_copy(ent_hbm.at[gxh.at[off]], hbufs[b], sem)
        pltpu.async_copy(ent_hbm.at[gxt.at[off]], tbufs[b], sem)
        pltpu.async_copy(relemb_hbm.at[gxr.at[off]], rbufs[b], sem)

    def drain(b):
        pltpu.make_async_copy(ent_hbm.at[gxh.at[pl.ds(0, CHUNK)]], hbufs[b],
                              sem).wait()
        pltpu.make_async_copy(ent_hbm.at[gxt.at[pl.ds(0, CHUNK)]], tbufs[b],
                              sem).wait()
        pltpu.make_async_copy(relemb_hbm.at[gxr.at[pl.ds(0, CHUNK)]],
                              rbufs[b], sem).wait()

    fire(0, 0)
    for ck in range(NCHUNK):
        b = ck & 1
        if ck + 1 < NCHUNK:
            fire(ck + 1, 1 - b)
        drain(b)
        hb, tb, rb = hbufs[b], tbufs[b], rbufs[b]
        off = ck * CHUNK

        def group_body(g, carry):
            goff = pl.multiple_of(off + g * 16, 16)
            slot = g * 16 + iota16
            hsel = (hidx[pl.ds(goff, 16)] & 1) * 64
            tsel = (tidx[pl.ds(goff, 16)] & 1) * 64
            rsel = (ridx[pl.ds(goff, 16)] & 1) * 64
            acc = jnp.zeros((16,), jnp.float32)
            for d in range(DIM):
                rot = (jnp.full((16,), d, jnp.int32) + iota16) & 63
                h = plsc.load_gather(hb, [slot, hsel + rot])
                t = plsc.load_gather(tb, [slot, tsel + rot])
                r = plsc.load_gather(rb, [slot, rsel + rot])
                acc = acc + h * t * r
            score = 1.0 / (1.0 + jnp.exp(-acc))
            oscr[pl.ds(goff, 16)] = score
            return carry

        lax.fori_loop(0, GPC, group_body, 0)

    pltpu.sync_copy(oscr, out_hbm.at[pl.ds(base, ROWS_PER_W)])


@functools.partial(
    pl.kernel,
    mesh=_MESH,
    out_type=jax.ShapeDtypeStruct((BATCH,), jnp.float32),
    compiler_params=_PARAMS,
    scratch_types=[
        pltpu.VMEM((ROWS_PER_W,), jnp.int32),
        pltpu.VMEM((ROWS_PER_W,), jnp.int32),
        pltpu.VMEM((ROWS_PER_W,), jnp.int32),
        pltpu.VMEM((ROWS_PER_W,), jnp.int32),
        pltpu.VMEM((ROWS_PER_W,), jnp.int32),
        pltpu.VMEM((ROWS_PER_W,), jnp.int32),
        pltpu.VMEM((CHUNK, 2 * DIM), jnp.float32),
        pltpu.VMEM((CHUNK, 2 * DIM), jnp.float32),
        pltpu.VMEM((CHUNK, 2 * DIM), jnp.float32),
        pltpu.VMEM((CHUNK, 2 * DIM), jnp.float32),
        pltpu.VMEM((CHUNK, 2 * DIM), jnp.float32),
        pltpu.VMEM((CHUNK, 2 * DIM), jnp.float32),
        pltpu.VMEM((ROWS_PER_W,), jnp.float32),
        pltpu.SemaphoreType.DMA,
    ],
)
def _gather_sc(*args):
    _gather_body(*args)


def kernel(head, tail, relation, entity_embed, relation_embed):
    ent2 = _transpose_sc(entity_embed.T, entity_embed[TAIL0:].T)
    rel2 = relation_embed.reshape(relation_embed.shape[0] // 2, 2 * DIM)
    return _gather_sc(head.astype(jnp.int32), tail.astype(jnp.int32),
                      relation.astype(jnp.int32), ent2, rel2)
